# baseline (speedup 1.0000x reference)
import functools

import jax
import jax.numpy as jnp
from jax import lax
from jax.experimental import pallas as pl
from jax.experimental.pallas import tpu as pltpu
from jax.experimental.pallas import tpu_sc as plsc

DIM = 64
NSAMP = 50
L = 16
NC = 2
NS = 16
NW = NC * NS
CHUNK = 16
CROWS = CHUNK * (1 + NSAMP)
NGRP = -(-NSAMP // L)


def _make_score_kernel(batch):
    b_per_w = batch // NW
    nchunk = b_per_w // CHUNK
    mesh = plsc.VectorSubcoreMesh(core_axis_name="c", subcore_axis_name="s")

    @functools.partial(
        pl.kernel,
        mesh=mesh,
        compiler_params=pltpu.CompilerParams(
            needs_layout_passes=False, use_tc_tiling_on_sc=False),
        out_type=jax.ShapeDtypeStruct((batch * NSAMP,), jnp.float32),
        scratch_types=[
            pltpu.VMEM((CROWS,), jnp.int32),
            pltpu.VMEM((CROWS,), jnp.int32),
            pltpu.VMEM((CROWS, DIM), jnp.float32),
            pltpu.VMEM((CROWS, DIM), jnp.float32),
            pltpu.VMEM((CHUNK * NSAMP + L,), jnp.float32),
            pltpu.VMEM((L * (L + 1),), jnp.float32),
            pltpu.SemaphoreType.DMA,
            pltpu.SemaphoreType.DMA,
        ],
    )
    def score_kernel(items_hbm, samples_hbm, emb_hbm, out_hbm, idx0, idx1,
                     rows0, rows1, out_v, tmp_v, sem0, sem1):
        wid = lax.axis_index("s") * NC + lax.axis_index("c")
        base = wid * b_per_w
        lane = lax.iota(jnp.int32, L)
        idx_b = (idx0, idx1)
        rows_b = (rows0, rows1)
        sem_b = (sem0, sem1)

        def fire(ci, b):
            cbase = base + ci * CHUNK
            pltpu.sync_copy(items_hbm.at[pl.ds(cbase, CHUNK)],
                            idx_b[b].at[pl.ds(0, CHUNK)])
            pltpu.sync_copy(
                samples_hbm.at[pl.ds(cbase * NSAMP, CHUNK * NSAMP)],
                idx_b[b].at[pl.ds(CHUNK, CHUNK * NSAMP)])
            pltpu.async_copy(emb_hbm.at[idx_b[b].at[pl.ds(0, CHUNK)]],
                             rows_b[b].at[pl.ds(0, CHUNK)], sem_b[b])
            pltpu.async_copy(
                emb_hbm.at[idx_b[b].at[pl.ds(CHUNK, CHUNK * NSAMP)]],
                rows_b[b].at[pl.ds(CHUNK, CHUNK * NSAMP)], sem_b[b])

        def drain(b):
            pltpu.make_async_copy(emb_hbm.at[pl.ds(0, CROWS)], rows_b[b],
                                  sem_b[b]).wait()

        def compute(ci, rows_v):
            def item_body(i, icarry):
                srow0 = CHUNK + i * NSAMP
                it = [rows_v[i, pl.ds(c * L, L)] for c in range(DIM // L)]
                tbase = lane * (L + 1)
                for g in (NGRP - 1, *range(NGRP - 1)):
                    for j in range(min(L, NSAMP - g * L)):
                        roff = srow0 + g * L + j
                        prods = [rows_v[roff, pl.ds(c * L, L)] * it[c]
                                 for c in range(DIM // L)]
                        tmp_v[pl.ds(j * (L + 1), L)] = (
                            (prods[0] + prods[1]) + (prods[2] + prods[3]))
                    cols = [plsc.load_gather(tmp_v, [tbase + d])
                            for d in range(L)]
                    while len(cols) > 1:
                        cols = [cols[k] + cols[k + 1]
                                for k in range(0, len(cols), 2)]
                    out_v[pl.ds(i * NSAMP + g * L, L)] = cols[0]
                return icarry

            lax.fori_loop(0, CHUNK, item_body, 0)
            cbase = base + ci * CHUNK
            pltpu.sync_copy(out_v.at[pl.ds(0, CHUNK * NSAMP)],
                            out_hbm.at[pl.ds(cbase * NSAMP, CHUNK * NSAMP)])

        fire(0, 0)

        def pair_body(cj, carry):
            for b in range(2):
                ci = cj * 2 + b
                nci = ci + 1

                @pl.when(nci < nchunk)
                def _():
                    fire(nci, 1 - b)

                drain(b)
                compute(ci, rows_b[b])
            return carry

        lax.fori_loop(0, nchunk // 2, pair_body, 0)

    return score_kernel


def kernel(items, samples, emb):
    batch = items.shape[0]
    items = items.reshape(batch).astype(jnp.int32)
    samples = samples.reshape(batch * NSAMP).astype(jnp.int32)
    znop = jnp.float32(items[0]) * jnp.float32(0.0)
    out = _make_score_kernel(batch)(items, samples, emb + znop)
    return out.reshape(batch, NSAMP)


# speedup vs baseline: 1.3738x; 1.3738x over previous
import functools

import jax
import jax.numpy as jnp
from jax import lax
from jax.experimental import pallas as pl
from jax.experimental.pallas import tpu as pltpu
from jax.experimental.pallas import tpu_sc as plsc

DIM = 64
NSAMP = 50
RPI = 1 + NSAMP
L = 16
NC = 2
NS = 16
NW = NC * NS
CHUNK = 16
CROWS = CHUNK * RPI
NGRP = -(-NSAMP // L)


def _make_score_kernel(batch):
    b_per_w = batch // NW
    nchunk = b_per_w // CHUNK
    mesh = plsc.VectorSubcoreMesh(core_axis_name="c", subcore_axis_name="s")

    @functools.partial(
        pl.kernel,
        mesh=mesh,
        compiler_params=pltpu.CompilerParams(
            needs_layout_passes=False, use_tc_tiling_on_sc=False),
        out_type=jax.ShapeDtypeStruct((batch * NSAMP,), jnp.float32),
        scratch_types=[
            pltpu.VMEM((CROWS,), jnp.int32),
            pltpu.VMEM((CROWS,), jnp.int32),
            pltpu.VMEM((CROWS, DIM), jnp.float32),
            pltpu.VMEM((CROWS, DIM), jnp.float32),
            pltpu.VMEM((CHUNK * NSAMP + L,), jnp.float32),
            pltpu.VMEM((L * (L + 1),), jnp.float32),
            pltpu.SemaphoreType.DMA,
            pltpu.SemaphoreType.DMA,
            pltpu.SemaphoreType.DMA,
            pltpu.SemaphoreType.DMA,
        ],
    )
    def score_kernel(idx_hbm, emb_hbm, out_hbm, idx0, idx1, rows0, rows1,
                     out_v, tmp_v, semr0, semr1, semi0, semi1):
        wid = lax.axis_index("s") * NC + lax.axis_index("c")
        base = wid * b_per_w
        lane = lax.iota(jnp.int32, L)
        idx_b = (idx0, idx1)
        rows_b = (rows0, rows1)
        semr_b = (semr0, semr1)
        semi_b = (semi0, semi1)

        def fire_idx(ci, b):
            pltpu.async_copy(idx_hbm.at[(base // CHUNK) + ci], idx_b[b],
                             semi_b[b])

        def wait_idx(b):
            pltpu.make_async_copy(idx_hbm.at[0], idx_b[b],
                                  semi_b[b]).wait()

        def fire_gather(b):
            pltpu.async_copy(emb_hbm.at[idx_b[b]], rows_b[b], semr_b[b])

        def drain_rows(b):
            pltpu.make_async_copy(emb_hbm.at[pl.ds(0, CROWS)], rows_b[b],
                                  semr_b[b]).wait()

        def compute(ci, rows_v):
            def item_body(i, icarry):
                row0 = i * RPI
                it = [rows_v[row0, pl.ds(c * L, L)] for c in range(DIM // L)]
                tbase = lane * (L + 1)
                for g in (NGRP - 1, *range(NGRP - 1)):
                    for j in range(min(L, NSAMP - g * L)):
                        roff = row0 + 1 + g * L + j
                        prods = [rows_v[roff, pl.ds(c * L, L)] * it[c]
                                 for c in range(DIM // L)]
                        tmp_v[pl.ds(j * (L + 1), L)] = (
                            (prods[0] + prods[1]) + (prods[2] + prods[3]))
                    cols = [plsc.load_gather(tmp_v, [tbase + d])
                            for d in range(L)]
                    while len(cols) > 1:
                        cols = [cols[k] + cols[k + 1]
                                for k in range(0, len(cols), 2)]
                    out_v[pl.ds(i * NSAMP + g * L, L)] = cols[0]
                return icarry

            lax.fori_loop(0, CHUNK, item_body, 0)
            cbase = base + ci * CHUNK
            pltpu.sync_copy(out_v.at[pl.ds(0, CHUNK * NSAMP)],
                            out_hbm.at[pl.ds(cbase * NSAMP, CHUNK * NSAMP)])

        fire_idx(0, 0)
        wait_idx(0)
        fire_gather(0)
        fire_idx(1, 1)

        def pair_body(cj, carry):
            for b in range(2):
                ci = cj * 2 + b
                nci = ci + 1

                @pl.when(nci < nchunk)
                def _():
                    wait_idx(1 - b)
                    fire_gather(1 - b)

                drain_rows(b)

                @pl.when(ci + 2 < nchunk)
                def _():
                    fire_idx(ci + 2, b)

                compute(ci, rows_b[b])
            return carry

        lax.fori_loop(0, nchunk // 2, pair_body, 0)

    return score_kernel


def kernel(items, samples, emb):
    batch = items.shape[0]
    idx = jnp.concatenate(
        [items.astype(jnp.int32).reshape(batch, 1),
         samples.astype(jnp.int32).reshape(batch, NSAMP)], axis=1)
    idx = idx.reshape(batch // CHUNK, CROWS)
    out = _make_score_kernel(batch)(idx, emb)
    return out.reshape(batch, NSAMP)


# speedup vs baseline: 1.3780x; 1.0031x over previous
import functools

import jax
import jax.numpy as jnp
from jax import lax
from jax.experimental import pallas as pl
from jax.experimental.pallas import tpu as pltpu
from jax.experimental.pallas import tpu_sc as plsc

DIM = 64
NSAMP = 50
RPI = 1 + NSAMP
L = 16
NC = 2
NS = 16
NW = NC * NS
CHUNK = 16
CROWS = CHUNK * RPI
NGRP = -(-NSAMP // L)


def _make_score_kernel(batch):
    b_per_w = batch // NW
    nchunk = b_per_w // CHUNK
    mesh = plsc.VectorSubcoreMesh(core_axis_name="c", subcore_axis_name="s")

    @functools.partial(
        pl.kernel,
        mesh=mesh,
        compiler_params=pltpu.CompilerParams(
            needs_layout_passes=False, use_tc_tiling_on_sc=False),
        out_type=jax.ShapeDtypeStruct((batch * NSAMP,), jnp.float32),
        scratch_types=[
            pltpu.VMEM((CROWS,), jnp.int32),
            pltpu.VMEM((CROWS,), jnp.int32),
            pltpu.VMEM((CROWS, DIM), jnp.float32),
            pltpu.VMEM((CROWS, DIM), jnp.float32),
            pltpu.VMEM((CHUNK * NSAMP + L,), jnp.float32),
            pltpu.VMEM((CHUNK * NSAMP + L,), jnp.float32),
            pltpu.VMEM((L * (L + 1),), jnp.float32),
            pltpu.SemaphoreType.DMA,
            pltpu.SemaphoreType.DMA,
            pltpu.SemaphoreType.DMA,
            pltpu.SemaphoreType.DMA,
            pltpu.SemaphoreType.DMA,
            pltpu.SemaphoreType.DMA,
        ],
    )
    def score_kernel(idx_hbm, emb_hbm, out_hbm, idx0, idx1, rows0, rows1,
                     outv0, outv1, tmp_v, semr0, semr1, semi0, semi1,
                     semo0, semo1):
        wid = lax.axis_index("s") * NC + lax.axis_index("c")
        base = wid * b_per_w
        lane = lax.iota(jnp.int32, L)
        idx_b = (idx0, idx1)
        rows_b = (rows0, rows1)
        semr_b = (semr0, semr1)
        semi_b = (semi0, semi1)
        out_b = (outv0, outv1)
        semo_b = (semo0, semo1)

        def fire_idx(ci, b):
            pltpu.async_copy(idx_hbm.at[(base // CHUNK) + ci], idx_b[b],
                             semi_b[b])

        def wait_idx(b):
            pltpu.make_async_copy(idx_hbm.at[0], idx_b[b],
                                  semi_b[b]).wait()

        def fire_gather(b):
            pltpu.async_copy(emb_hbm.at[idx_b[b]], rows_b[b], semr_b[b])

        def drain_rows(b):
            pltpu.make_async_copy(emb_hbm.at[pl.ds(0, CROWS)], rows_b[b],
                                  semr_b[b]).wait()

        def wait_out(b):
            pltpu.make_async_copy(out_hbm.at[pl.ds(0, CHUNK * NSAMP)],
                                  out_b[b].at[pl.ds(0, CHUNK * NSAMP)],
                                  semo_b[b]).wait()

        def compute(ci, b, rows_v):
            out_v = out_b[b]

            @pl.when(ci >= 2)
            def _():
                wait_out(b)

            def item_body(i, icarry):
                row0 = i * RPI
                it = [rows_v[row0, pl.ds(c * L, L)] for c in range(DIM // L)]
                tbase = lane * (L + 1)
                for g in (NGRP - 1, *range(NGRP - 1)):
                    for j in range(min(L, NSAMP - g * L)):
                        roff = row0 + 1 + g * L + j
                        prods = [rows_v[roff, pl.ds(c * L, L)] * it[c]
                                 for c in range(DIM // L)]
                        tmp_v[pl.ds(j * (L + 1), L)] = (
                            (prods[0] + prods[1]) + (prods[2] + prods[3]))
                    cols = [plsc.load_gather(tmp_v, [tbase + d])
                            for d in range(L)]
                    while len(cols) > 1:
                        cols = [cols[k] + cols[k + 1]
                                for k in range(0, len(cols), 2)]
                    out_v[pl.ds(i * NSAMP + g * L, L)] = cols[0]
                return icarry

            lax.fori_loop(0, CHUNK, item_body, 0)
            cbase = base + ci * CHUNK
            pltpu.async_copy(out_v.at[pl.ds(0, CHUNK * NSAMP)],
                             out_hbm.at[pl.ds(cbase * NSAMP, CHUNK * NSAMP)],
                             semo_b[b])

        fire_idx(0, 0)
        wait_idx(0)
        fire_gather(0)
        fire_idx(1, 1)

        def pair_body(cj, carry):
            for b in range(2):
                ci = cj * 2 + b
                nci = ci + 1

                @pl.when(nci < nchunk)
                def _():
                    wait_idx(1 - b)
                    fire_gather(1 - b)

                drain_rows(b)

                @pl.when(ci + 2 < nchunk)
                def _():
                    fire_idx(ci + 2, b)

                compute(ci, b, rows_b[b])
            return carry

        lax.fori_loop(0, nchunk // 2, pair_body, 0)
        wait_out(0)
        wait_out(1)

    return score_kernel


def kernel(items, samples, emb):
    batch = items.shape[0]
    idx = jnp.concatenate(
        [items.astype(jnp.int32).reshape(batch, 1),
         samples.astype(jnp.int32).reshape(batch, NSAMP)], axis=1)
    idx = idx.reshape(batch // CHUNK, CROWS)
    out = _make_score_kernel(batch)(idx, emb)
    return out.reshape(batch, NSAMP)
